# Initial kernel scaffold; baseline (speedup 1.0000x reference)
import jax
import jax.numpy as jnp
from jax.experimental import pallas as pl


def kernel(x, pos, edge_index):
    raise NotImplementedError("write your pallas kernel here")


# speedup vs baseline: 1.4936x; 1.4936x over previous
import jax
import jax.numpy as jnp
from jax import lax
from jax.experimental import pallas as pl
from jax.experimental.pallas import tpu as pltpu
from jax.experimental.pallas import tpu_sc as plsc

N_NODES = 10000
D_FEAT = 128
D_OUT = 131
D_PAD = 144
N_W = 32
NPW = 320
N_PAD = N_W * NPW
W_EDGES = 8000
C_GATH = 128


def _scalar(v, e, lane):
    return jnp.max(jnp.where(lane == e, v, 0))


def _sc_body(xp_hbm, pospad_hbm, src_hbm, dst_hbm, out_hbm,
             acc, dstw, srcw, cls, cll, rows, posd, sem):
    n_edges = src_hbm.shape[0]
    nc = 2
    wid = lax.axis_index("s") * nc + lax.axis_index("c")
    lo = wid * NPW
    lane = lax.broadcasted_iota(jnp.int32, (16,), 0)

    pltpu.sync_copy(xp_hbm.at[pl.ds(lo, NPW)], acc.at[pl.ds(0, NPW)])
    zf = jnp.zeros((16,), jnp.float32)
    for c in range(D_PAD // 16):
        acc[NPW, pl.ds(c * 16, 16)] = zf
    pltpu.sync_copy(pospad_hbm.at[pl.ds(lo, NPW)], posd)

    def window(w, _):
        base = w * W_EDGES
        pltpu.sync_copy(dst_hbm.at[pl.ds(base, W_EDGES)], dstw)
        pltpu.sync_copy(src_hbm.at[pl.ds(base, W_EDGES)], srcw)

        def filt(i, cnt_v):
            d = dstw[pl.ds(i * 16, 16)]
            s = srcw[pl.ds(i * 16, 16)]
            dl = d - lo
            m = dl.astype(jnp.uint32) < jnp.uint32(NPW)
            ps = plsc.cumsum(jnp.where(m, 1, 0))
            idx = cnt_v + ps - 1
            plsc.store_scatter(cls, [idx], s, mask=m)
            plsc.store_scatter(cll, [idx], dl, mask=m)
            npop = plsc.all_reduce_population_count(m)
            npop = npop if getattr(npop, "ndim", 0) else jnp.full(
                (16,), npop, jnp.int32)
            return cnt_v + npop

        cnt_v = lax.fori_loop(0, W_EDGES // 16, filt,
                              jnp.zeros((16,), jnp.int32))
        cnt = jnp.max(cnt_v)

        for t in range(C_GATH // 16):
            tidx = cnt_v + (t * 16) + lane
            plsc.store_scatter(cls, [tidx], jnp.zeros((16,), jnp.int32))
            plsc.store_scatter(cll, [tidx], jnp.full((16,), NPW, jnp.int32))

        def chunk(ci, _):
            cb = pl.multiple_of(ci * C_GATH, 8)
            pltpu.async_copy(xp_hbm.at[cls.at[pl.ds(cb, C_GATH)]], rows,
                             sem).wait()

            def group(g, _):
                ldv = cll[pl.ds(pl.multiple_of(cb + g * 16, 8), 16)]
                for e in range(16):
                    ld = _scalar(ldv, e, lane)
                    r = g * 16 + e
                    for c in range(D_PAD // 16):
                        sl = pl.ds(c * 16, 16)
                        acc[ld, sl] = jnp.maximum(acc[ld, sl], rows[r, sl])
                return 0

            lax.fori_loop(0, C_GATH // 16, group, 0)
            return 0

        nch = lax.shift_right_logical(cnt + (C_GATH - 1), 7)
        lax.fori_loop(0, nch, chunk, 0)
        return 0

    lax.fori_loop(0, n_edges // W_EDGES, window, 0)

    def fin(r, _):
        acc[r, pl.ds(128, 16)] = acc[r, pl.ds(128, 16)] - posd[r, :]
        return 0

    lax.fori_loop(0, NPW, fin, 0)
    pltpu.sync_copy(acc.at[pl.ds(0, NPW)], out_hbm.at[pl.ds(lo, NPW)])


@jax.jit
def _pointconv_sc(xp, pospad, src, dst):
    mesh = plsc.VectorSubcoreMesh(core_axis_name="c", subcore_axis_name="s")
    f = pl.kernel(
        _sc_body, mesh=mesh,
        out_type=jax.ShapeDtypeStruct((N_PAD, D_PAD), jnp.float32),
        compiler_params=pltpu.CompilerParams(
            use_tc_tiling_on_sc=False, needs_layout_passes=False),
        scratch_types=[
            pltpu.VMEM((NPW + 1, D_PAD), jnp.float32),
            pltpu.VMEM((W_EDGES,), jnp.int32),
            pltpu.VMEM((W_EDGES,), jnp.int32),
            pltpu.VMEM((W_EDGES + C_GATH,), jnp.int32),
            pltpu.VMEM((W_EDGES + C_GATH,), jnp.int32),
            pltpu.VMEM((C_GATH, D_PAD), jnp.float32),
            pltpu.VMEM((NPW, 16), jnp.float32),
            pltpu.SemaphoreType.DMA,
        ],
    )
    return f(xp, pospad, src, dst)


def kernel(x, pos, edge_index):
    n = x.shape[0]
    src = edge_index[0].astype(jnp.int32)
    dst = edge_index[1].astype(jnp.int32)
    xp = jnp.concatenate(
        [x, pos, jnp.zeros((n, D_PAD - D_OUT), jnp.float32)], axis=1)
    xp = jnp.concatenate(
        [xp, jnp.zeros((N_PAD - n, D_PAD), jnp.float32)], axis=0)
    pospad = jnp.concatenate([pos, jnp.zeros((n, 13), jnp.float32)], axis=1)
    pospad = jnp.concatenate(
        [pospad, jnp.zeros((N_PAD - n, 16), jnp.float32)], axis=0)
    out = _pointconv_sc(xp, pospad, src, dst)
    return out[:n, :D_OUT]
